# Initial kernel scaffold; baseline (speedup 1.0000x reference)
import jax
import jax.numpy as jnp
from jax.experimental import pallas as pl


def kernel(x, edge_index, batch, W_gat, a_src, a_dst, b_gat, W_gcn, b_gcn, W_fc1, b_fc1, W_fc2, b_fc2):
    raise NotImplementedError("write your pallas kernel here")


# speedup vs baseline: 1.0595x; 1.0595x over previous
import functools

import jax
import jax.numpy as jnp
from jax.experimental import pallas as pl
from jax.experimental.pallas import tpu as pltpu

N_NODES = 50000
N_EDGES = 800000
D_IN = 78
HEADS = 10
D_HID = D_IN * HEADS
N_GRAPHS = 512


def _mlp_body(g_ref, w1_ref, b1_ref, w2_ref, b2_ref, out_ref):
    g = g_ref[...]
    t = jnp.maximum(jnp.dot(g, w1_ref[...], preferred_element_type=jnp.float32)
                    + b1_ref[...][None, :], 0.0)
    out_ref[...] = (jnp.dot(t, w2_ref[...], preferred_element_type=jnp.float32)
                    + b2_ref[...][None, :])


def _mlp(g, W_fc1, b_fc1, W_fc2, b_fc2):
    return pl.pallas_call(
        _mlp_body,
        out_shape=jax.ShapeDtypeStruct((g.shape[0], W_fc2.shape[1]), jnp.float32),
    )(g, W_fc1, b_fc1, W_fc2, b_fc2)


def kernel(x, edge_index, batch, W_gat, a_src, a_dst, b_gat, W_gcn, b_gcn,
           W_fc1, b_fc1, W_fc2, b_fc2):
    n = x.shape[0]
    src = edge_index[0].astype(jnp.int32)
    dst = edge_index[1].astype(jnp.int32)
    loops = jnp.arange(n, dtype=jnp.int32)
    src = jnp.concatenate([src, loops])
    dst = jnp.concatenate([dst, loops])

    h = x @ W_gat
    As = jnp.einsum("dhk,hk->dh", W_gat.reshape(D_IN, HEADS, D_IN), a_src)
    Ad = jnp.einsum("dhk,hk->dh", W_gat.reshape(D_IN, HEADS, D_IN), a_dst)
    als = x @ As
    ald = x @ Ad

    e = als[src] + ald[dst]
    e = jnp.where(e >= 0, e, 0.2 * e)
    w = jnp.exp(e)
    denom = jax.ops.segment_sum(w, dst, num_segments=n)
    msg = h.reshape(n, HEADS, D_IN)[src] * w[:, :, None]
    num = jax.ops.segment_sum(msg, dst, num_segments=n).reshape(n, D_HID)
    gat = num / (jnp.repeat(denom, D_IN, axis=1) + 1e-16) + b_gat
    gat = jnp.maximum(gat, 0.0)

    deg = jax.ops.segment_sum(jnp.ones_like(dst, dtype=jnp.float32), dst,
                              num_segments=n)
    dinv = jnp.where(deg > 0, jax.lax.rsqrt(jnp.maximum(deg, 1e-12)), 0.0)
    h2 = gat @ W_gcn
    msg2 = h2[src] * (dinv[src] * dinv[dst])[:, None]
    out2 = jax.ops.segment_sum(msg2, dst, num_segments=n) + b_gcn
    out2 = jnp.maximum(out2, 0.0)

    bi = batch.astype(jnp.int32)
    sums = jax.ops.segment_sum(out2, bi, num_segments=N_GRAPHS)
    cnt = jax.ops.segment_sum(jnp.ones((n,), jnp.float32), bi,
                              num_segments=N_GRAPHS)
    mean = sums / jnp.maximum(cnt, 1.0)[:, None]
    g = jnp.concatenate([mean, sums], axis=1)
    return _mlp(g, W_fc1, b_fc1, W_fc2, b_fc2)


# speedup vs baseline: 1.7341x; 1.6367x over previous
import jax
import jax.numpy as jnp
from jax import lax
from jax.experimental import pallas as pl
from jax.experimental.pallas import tpu as pltpu
from jax.experimental.pallas import tpu_sc as plsc

N_NODES = 50000
N_EDGES = 800000
D_IN = 78
HEADS = 10
D_HID = D_IN * HEADS
N_GRAPHS = 512

NP = 51200
DP = 896
NB = DP // 16
BPH = 5
NW = 128
WIN = 64
CH = 16 * WIN
NCH = NP // CH
PIECE = 2000
NPIECES = N_EDGES // PIECE
IDXROWS = (PIECE + 32) // 16

_F32 = jnp.float32
_I32 = jnp.int32

_B2H = [min(j // BPH, 10) for j in range(NB)]


def _mesh():
    return plsc.VectorSubcoreMesh(core_axis_name="c", subcore_axis_name="s")


def _prep_body(x_ref, w_ref, aa_ref, h_ref, alo_ref):
    xb = x_ref[...]
    h_ref[...] = jnp.dot(xb, w_ref[...], preferred_element_type=_F32)
    alo_ref[...] = jnp.dot(xb, aa_ref[...], preferred_element_type=_F32)


def _prep(x_pad, w_pad, aa_pad):
    nblk = NP // 800
    return pl.pallas_call(
        _prep_body,
        grid=(nblk,),
        in_specs=[
            pl.BlockSpec((800, D_IN), lambda i: (i, 0)),
            pl.BlockSpec((D_IN, DP), lambda i: (0, 0)),
            pl.BlockSpec((D_IN, NW), lambda i: (0, 0)),
        ],
        out_specs=[
            pl.BlockSpec((800, DP), lambda i: (i, 0)),
            pl.BlockSpec((800, NW), lambda i: (i, 0)),
        ],
        out_shape=[
            jax.ShapeDtypeStruct((NP, DP), _F32),
            jax.ShapeDtypeStruct((NP, NW), _F32),
        ],
    )(x_pad, w_pad, aa_pad)


def _gat_sc_body(src_h, dst_h, h_h, alo_h, bias_h,
                 gatr_h, denr_h,
                 acc, dacc, alw, eps0, epd0, eps1, epd1, cols, cold2, rows,
                 alsb, biasv, hbt, wbuf,
                 s0, s1, s2, s3, s4, s5):
    c = lax.axis_index("c")
    s = lax.axis_index("s")
    lane = lax.iota(_I32, 16)
    pltpu.sync_copy(bias_h, biasv)
    epsb = [eps0, eps1]
    epdb = [epd0, epd1]
    ssem = [s0, s1]
    dsem = [s2, s3]

    @pl.loop(0, IDXROWS)
    def _ci(i):
        cols[pl.ds(16 * i, 16)] = lane

    for j in range(NB):
        hbt[j, pl.ds(0, 16)] = jnp.full((16,), _B2H[j], _I32)

    @pl.loop(0, NCH // 2)
    def _chunk(ci):
        w0 = pl.multiple_of(((2 * ci + c) * CH + s * WIN), 16)

        d1 = pltpu.async_copy(h_h.at[pl.ds(w0, WIN)], acc, s4)
        d2 = pltpu.async_copy(alo_h.at[pl.ds(w0, WIN)], alw, s5)
        d1.wait()
        d2.wait()

        @pl.loop(0, WIN)
        def _ir(r):
            e = alw[r, pl.ds(0, 16)] + alw[r, pl.ds(16, 16)]
            w = jnp.exp(jnp.where(e >= 0.0, e, 0.2 * e))
            dacc[r, pl.ds(0, 16)] = w
            for j in range(NB):
                wj = jnp.broadcast_to(w[_B2H[j]], (16,))
                acc[r, pl.ds(16 * j, 16)] = acc[r, pl.ds(16 * j, 16)] * wj

        pltpu.async_copy(src_h.at[pl.ds(0, PIECE)], epsb[0], ssem[0])
        pltpu.async_copy(dst_h.at[pl.ds(0, PIECE)], epdb[0], dsem[0])

        @pl.loop(0, NPIECES // 2)
        def _pp(pp):
            for b in range(2):
                pltpu.make_async_copy(
                    src_h.at[pl.ds(0, PIECE)], epsb[b], ssem[b]).wait()
                pltpu.make_async_copy(
                    dst_h.at[pl.ds(0, PIECE)], epdb[b], dsem[b]).wait()
                nxt = 2 * pp + b + 1

                @pl.when(nxt < NPIECES)
                def _():
                    off = pl.multiple_of(nxt * PIECE, PIECE)
                    pltpu.async_copy(
                        src_h.at[pl.ds(off, PIECE)], epsb[1 - b],
                        ssem[1 - b])
                    pltpu.async_copy(
                        dst_h.at[pl.ds(off, PIECE)], epdb[1 - b],
                        dsem[1 - b])

                eps = epsb[b]
                epd = epdb[b]

                @pl.loop(0, PIECE // 16, init_carry=jnp.int32(0))
                def _scan(v, k):
                    rel = epd[pl.ds(16 * v, 16)] - w0
                    m = (rel >= 0) & (rel < WIN)
                    csum = plsc.cumsum(jnp.where(m, 1, 0).astype(_I32))
                    pos = k + csum - 1
                    plsc.store_scatter(cols, [pos], eps[pl.ds(16 * v, 16)],
                                       mask=m)
                    plsc.store_scatter(cold2, [pos >> 4, pos & 15], rel,
                                       mask=m)
                    return k + csum[15]

                k = _scan
                ng = (k + 15) // 16

                @pl.loop(0, ng)
                def _grp(g):
                    s16 = cols[pl.ds(16 * g, 16)]
                    d1 = pltpu.async_copy(alo_h.at[s16], alsb, s4)
                    d2 = pltpu.async_copy(h_h.at[s16], rows, s5)
                    d1.wait()
                    d2.wait()
                    rel16 = cold2[g, pl.ds(0, 16)]

                    for r in range(16):
                        @pl.when(16 * g + r < k)
                        def _edge(r=r):
                            rel = rel16[r]
                            e = (alsb[r, pl.ds(0, 16)]
                                 + alw[rel, pl.ds(16, 16)])
                            w = jnp.exp(jnp.where(e >= 0.0, e, 0.2 * e))
                            dacc[rel, pl.ds(0, 16)] = (
                                dacc[rel, pl.ds(0, 16)] + w)
                            wbuf[pl.ds(0, 16)] = w

                            @pl.loop(0, NB)
                            def _blk(j):
                                hv = hbt[j, pl.ds(0, 16)]
                                wj = plsc.load_gather(wbuf, [hv])
                                acc[rel, pl.ds(16 * j, 16)] = (
                                    acc[rel, pl.ds(16 * j, 16)]
                                    + rows[r, pl.ds(16 * j, 16)] * wj)

        @pl.loop(0, WIN)
        def _fr(r):
            d = dacc[r, pl.ds(0, 16)]
            for j in range(NB):
                dj = jnp.broadcast_to(d[_B2H[j]], (16,))
                v = (acc[r, pl.ds(16 * j, 16)] / dj
                     + biasv[pl.ds(16 * j, 16)])
                acc[r, pl.ds(16 * j, 16)] = jnp.maximum(v, 0.0)

        d1 = pltpu.async_copy(acc, gatr_h.at[pl.ds(w0, WIN)], s4)
        d2 = pltpu.async_copy(dacc, denr_h.at[pl.ds(w0, WIN)], s5)
        d1.wait()
        d2.wait()


def _gat_sc(src, dst, h, alo, bias_pad):
    f = pl.kernel(
        _gat_sc_body,
        out_type=[
            jax.ShapeDtypeStruct((NP, DP), _F32),
            jax.ShapeDtypeStruct((NP, 16), _F32),
        ],
        mesh=_mesh(),
        compiler_params=pltpu.CompilerParams(needs_layout_passes=False),
        scratch_types=[
            pltpu.VMEM((WIN, DP), _F32),
            pltpu.VMEM((WIN, 16), _F32),
            pltpu.VMEM((WIN, NW), _F32),
            pltpu.VMEM((PIECE,), _I32),
            pltpu.VMEM((PIECE,), _I32),
            pltpu.VMEM((PIECE,), _I32),
            pltpu.VMEM((PIECE,), _I32),
            pltpu.VMEM((PIECE + 32,), _I32),
            pltpu.VMEM((IDXROWS, 16), _I32),
            pltpu.VMEM((16, DP), _F32),
            pltpu.VMEM((16, NW), _F32),
            pltpu.VMEM((DP,), _F32),
            pltpu.VMEM((NB, 16), _I32),
            pltpu.VMEM((16,), _F32),
            pltpu.SemaphoreType.DMA,
            pltpu.SemaphoreType.DMA,
            pltpu.SemaphoreType.DMA,
            pltpu.SemaphoreType.DMA,
            pltpu.SemaphoreType.DMA,
            pltpu.SemaphoreType.DMA,
        ],
    )
    return f(src, dst, h, alo, bias_pad)


def _gcn_mm_body(g_ref, w_ref, den_ref, h2_ref, dinv_ref):
    h2_ref[...] = jnp.dot(g_ref[...], w_ref[...], preferred_element_type=_F32)
    deg = den_ref[...][:, 10:11]
    dinv_ref[...] = jnp.broadcast_to(lax.rsqrt(deg), (800, NW))


def _gcn_mm(gatr, w_gcn_pad, denr):
    nblk = NP // 800
    return pl.pallas_call(
        _gcn_mm_body,
        grid=(nblk,),
        in_specs=[
            pl.BlockSpec((800, DP), lambda i: (i, 0)),
            pl.BlockSpec((DP, DP), lambda i: (0, 0)),
            pl.BlockSpec((800, 16), lambda i: (i, 0)),
        ],
        out_specs=[
            pl.BlockSpec((800, DP), lambda i: (i, 0)),
            pl.BlockSpec((800, NW), lambda i: (i, 0)),
        ],
        out_shape=[
            jax.ShapeDtypeStruct((NP, DP), _F32),
            jax.ShapeDtypeStruct((NP, NW), _F32),
        ],
    )(gatr, w_gcn_pad, denr)


def _gcn_sc_body(src_h, dst_h, h2_h, dinv_h, bias_h,
                 h3_h,
                 acc, dw, eps0, epd0, eps1, epd1, cols, cold2, rows, dsb,
                 biasv,
                 s0, s1, s2, s3, s4, s5):
    c = lax.axis_index("c")
    s = lax.axis_index("s")
    lane = lax.iota(_I32, 16)
    pltpu.sync_copy(bias_h, biasv)
    epsb = [eps0, eps1]
    epdb = [epd0, epd1]
    ssem = [s0, s1]
    dsem = [s2, s3]

    @pl.loop(0, IDXROWS)
    def _ci(i):
        cols[pl.ds(16 * i, 16)] = lane

    @pl.loop(0, NCH // 2)
    def _chunk(ci):
        w0 = pl.multiple_of(((2 * ci + c) * CH + s * WIN), 16)

        d1 = pltpu.async_copy(h2_h.at[pl.ds(w0, WIN)], acc, s4)
        d2 = pltpu.async_copy(dinv_h.at[pl.ds(w0, WIN)], dw, s5)
        d1.wait()
        d2.wait()

        @pl.loop(0, WIN)
        def _ir(r):
            nv = dw[r, pl.ds(0, 16)]
            nv2 = nv * nv
            for j in range(NB):
                acc[r, pl.ds(16 * j, 16)] = acc[r, pl.ds(16 * j, 16)] * nv2

        pltpu.async_copy(src_h.at[pl.ds(0, PIECE)], epsb[0], ssem[0])
        pltpu.async_copy(dst_h.at[pl.ds(0, PIECE)], epdb[0], dsem[0])

        @pl.loop(0, NPIECES // 2)
        def _pp(pp):
            for b in range(2):
                pltpu.make_async_copy(
                    src_h.at[pl.ds(0, PIECE)], epsb[b], ssem[b]).wait()
                pltpu.make_async_copy(
                    dst_h.at[pl.ds(0, PIECE)], epdb[b], dsem[b]).wait()
                nxt = 2 * pp + b + 1

                @pl.when(nxt < NPIECES)
                def _():
                    off = pl.multiple_of(nxt * PIECE, PIECE)
                    pltpu.async_copy(
                        src_h.at[pl.ds(off, PIECE)], epsb[1 - b],
                        ssem[1 - b])
                    pltpu.async_copy(
                        dst_h.at[pl.ds(off, PIECE)], epdb[1 - b],
                        dsem[1 - b])

                eps = epsb[b]
                epd = epdb[b]

                @pl.loop(0, PIECE // 16, init_carry=jnp.int32(0))
                def _scan(v, k):
                    rel = epd[pl.ds(16 * v, 16)] - w0
                    m = (rel >= 0) & (rel < WIN)
                    csum = plsc.cumsum(jnp.where(m, 1, 0).astype(_I32))
                    pos = k + csum - 1
                    plsc.store_scatter(cols, [pos], eps[pl.ds(16 * v, 16)],
                                       mask=m)
                    plsc.store_scatter(cold2, [pos >> 4, pos & 15], rel,
                                       mask=m)
                    return k + csum[15]

                k = _scan
                ng = (k + 15) // 16

                @pl.loop(0, ng)
                def _grp(g):
                    s16 = cols[pl.ds(16 * g, 16)]
                    d1 = pltpu.async_copy(dinv_h.at[s16], dsb, s4)
                    d2 = pltpu.async_copy(h2_h.at[s16], rows, s5)
                    d1.wait()
                    d2.wait()
                    rel16 = cold2[g, pl.ds(0, 16)]

                    for r in range(16):
                        @pl.when(16 * g + r < k)
                        def _edge(r=r):
                            rel = rel16[r]
                            nv = (dsb[r, pl.ds(0, 16)]
                                  * dw[rel, pl.ds(0, 16)])

                            @pl.loop(0, NB)
                            def _blk(j):
                                acc[rel, pl.ds(16 * j, 16)] = (
                                    acc[rel, pl.ds(16 * j, 16)]
                                    + rows[r, pl.ds(16 * j, 16)] * nv)

        @pl.loop(0, WIN)
        def _fr(r):
            for j in range(NB):
                v = acc[r, pl.ds(16 * j, 16)] + biasv[pl.ds(16 * j, 16)]
                acc[r, pl.ds(16 * j, 16)] = jnp.maximum(v, 0.0)

        pltpu.async_copy(acc, h3_h.at[pl.ds(w0, WIN)], s4).wait()


def _gcn_sc(src, dst, h2, dinv, bias_pad):
    f = pl.kernel(
        _gcn_sc_body,
        out_type=jax.ShapeDtypeStruct((NP, DP), _F32),
        mesh=_mesh(),
        compiler_params=pltpu.CompilerParams(needs_layout_passes=False),
        scratch_types=[
            pltpu.VMEM((WIN, DP), _F32),
            pltpu.VMEM((WIN, NW), _F32),
            pltpu.VMEM((PIECE,), _I32),
            pltpu.VMEM((PIECE,), _I32),
            pltpu.VMEM((PIECE,), _I32),
            pltpu.VMEM((PIECE,), _I32),
            pltpu.VMEM((PIECE + 32,), _I32),
            pltpu.VMEM((IDXROWS, 16), _I32),
            pltpu.VMEM((16, DP), _F32),
            pltpu.VMEM((16, NW), _F32),
            pltpu.VMEM((DP,), _F32),
            pltpu.SemaphoreType.DMA,
            pltpu.SemaphoreType.DMA,
            pltpu.SemaphoreType.DMA,
            pltpu.SemaphoreType.DMA,
            pltpu.SemaphoreType.DMA,
            pltpu.SemaphoreType.DMA,
        ],
    )
    return f(src, dst, h2, dinv, bias_pad)


def _pool_body(b_ref, h3_ref, pool_ref, cnt_ref):
    i = pl.program_id(0)

    @pl.when(i == 0)
    def _():
        pool_ref[...] = jnp.zeros_like(pool_ref)
        cnt_ref[...] = jnp.zeros_like(cnt_ref)

    bb = b_ref[...]
    gi = lax.broadcasted_iota(_I32, (1, N_GRAPHS), 1)
    oh = (bb == gi).astype(_F32)
    pool_ref[...] += lax.dot_general(
        oh, h3_ref[...], dimension_numbers=(((0,), (0,)), ((), ())),
        preferred_element_type=_F32)
    cnt_ref[...] += jnp.broadcast_to(
        jnp.sum(oh, axis=0)[:, None], (N_GRAPHS, NW))


def _pool(batch2d, h3):
    nblk = NP // 800
    return pl.pallas_call(
        _pool_body,
        grid=(nblk,),
        in_specs=[
            pl.BlockSpec((800, 1), lambda i: (i, 0)),
            pl.BlockSpec((800, DP), lambda i: (i, 0)),
        ],
        out_specs=[
            pl.BlockSpec((N_GRAPHS, DP), lambda i: (0, 0)),
            pl.BlockSpec((N_GRAPHS, NW), lambda i: (0, 0)),
        ],
        out_shape=[
            jax.ShapeDtypeStruct((N_GRAPHS, DP), _F32),
            jax.ShapeDtypeStruct((N_GRAPHS, NW), _F32),
        ],
    )(batch2d, h3)


def _mlp_body(p_ref, c_ref, w1_ref, b1_ref, w2_ref, b2_ref, out_ref):
    sums = p_ref[...]
    cnt = c_ref[...][:, 0:1]
    mean = sums / jnp.maximum(cnt, 1.0)
    g = jnp.concatenate([mean, sums], axis=1)
    t = jnp.maximum(
        jnp.dot(g, w1_ref[...], preferred_element_type=_F32)
        + b1_ref[...][None, :], 0.0)
    out_ref[...] = (jnp.dot(t, w2_ref[...], preferred_element_type=_F32)
                    + b2_ref[...][None, :])


def _mlp(ps, cnt, W_fc1, b_fc1, W_fc2, b_fc2):
    return pl.pallas_call(
        _mlp_body,
        out_shape=jax.ShapeDtypeStruct((N_GRAPHS, W_fc2.shape[1]), _F32),
    )(ps, cnt, W_fc1, b_fc1, W_fc2, b_fc2)


def _head_pad(v):
    v800 = jnp.pad(v.reshape(HEADS, D_IN), ((0, 0), (0, 2))).reshape(800)
    return jnp.pad(v800, (0, DP - 800))


def kernel(x, edge_index, batch, W_gat, a_src, a_dst, b_gat, W_gcn, b_gcn,
           W_fc1, b_fc1, W_fc2, b_fc2):
    src = edge_index[0].astype(_I32)
    dst = edge_index[1].astype(_I32)

    w3 = W_gat.reshape(D_IN, HEADS, D_IN)
    As = jnp.einsum("dhk,hk->dh", w3, a_src)
    Ad = jnp.einsum("dhk,hk->dh", w3, a_dst)
    aa_pad = jnp.zeros((D_IN, NW), _F32)
    aa_pad = aa_pad.at[:, 0:HEADS].set(As).at[:, 16:16 + HEADS].set(Ad)
    w_pad = jnp.pad(
        jnp.pad(w3, ((0, 0), (0, 0), (0, 2))).reshape(D_IN, 800),
        ((0, 0), (0, DP - 800)))
    wg4 = W_gcn.reshape(HEADS, D_IN, HEADS, D_IN)
    wg800 = jnp.pad(wg4, ((0, 0), (0, 2), (0, 0), (0, 2))).reshape(800, 800)
    wg_pad = jnp.pad(wg800, ((0, DP - 800), (0, DP - 800)))
    bgat_pad = _head_pad(b_gat)
    bgcn_pad = _head_pad(b_gcn)
    x_pad = jnp.pad(x, ((0, NP - N_NODES), (0, 0)))
    batch2d = jnp.concatenate([
        batch.astype(_I32),
        jnp.full((NP - N_NODES,), N_GRAPHS, _I32)]).reshape(NP, 1)

    h, alo = _prep(x_pad, w_pad, aa_pad)
    gatr, denr = _gat_sc(src, dst, h, alo, bgat_pad)
    h2, dinv = _gcn_mm(gatr, wg_pad, denr)
    h3 = _gcn_sc(src, dst, h2, dinv, bgcn_pad)
    pool, cnt = _pool(batch2d, h3)

    ps = pool[:, :800].reshape(N_GRAPHS, HEADS, D_IN + 2)
    ps = ps[:, :, :D_IN].reshape(N_GRAPHS, D_HID)
    return _mlp(ps, cnt, W_fc1, b_fc1, W_fc2, b_fc2)


# speedup vs baseline: 2.0686x; 1.1929x over previous
import jax
import jax.numpy as jnp
from jax import lax
from jax.experimental import pallas as pl
from jax.experimental.pallas import tpu as pltpu
from jax.experimental.pallas import tpu_sc as plsc

N_NODES = 50000
N_EDGES = 800000
D_IN = 78
HEADS = 10
D_HID = D_IN * HEADS
N_GRAPHS = 512

NP = 51200
DP = 896
NB = DP // 16
BPH = 5
NW = 128
WIN = 80
CH = 16 * WIN
NCH = NP // CH
PIECE = 2000
BIASW = 1024
NPIECES = N_EDGES // PIECE
IDXROWS = (PIECE + 32) // 16

_F32 = jnp.float32
_I32 = jnp.int32

NBD = HEADS * BPH
_B2H = [j // BPH for j in range(NBD)]


def _mesh():
    return plsc.VectorSubcoreMesh(core_axis_name="c", subcore_axis_name="s")


def _prep_body(x_ref, w_ref, aa_ref, h_ref):
    xb = x_ref[...]
    h_ref[...] = jnp.dot(xb, w_ref[...], preferred_element_type=_F32)
    al = jnp.dot(xb, aa_ref[...], preferred_element_type=_F32)
    h_ref[:, 800:832] = al[:, 0:32]


def _prep(x_pad, w_pad, aa_pad):
    nblk = NP // 800
    return pl.pallas_call(
        _prep_body,
        grid=(nblk,),
        in_specs=[
            pl.BlockSpec((800, D_IN), lambda i: (i, 0)),
            pl.BlockSpec((D_IN, DP), lambda i: (0, 0)),
            pl.BlockSpec((D_IN, NW), lambda i: (0, 0)),
        ],
        out_specs=pl.BlockSpec((800, DP), lambda i: (i, 0)),
        out_shape=jax.ShapeDtypeStruct((NP, DP), _F32),
    )(x_pad, w_pad, aa_pad)


def _gat_sc_body(src_h, dst_h, h_h, bias_h,
                 gatr_h, denr_h,
                 acc, dacc, eps0, epd0, eps1, epd1, cols, cold2, rows,
                 biasv, hbt,
                 s0, s1, s2, s3, s4, s5):
    c = lax.axis_index("c")
    s = lax.axis_index("s")
    lane = lax.iota(_I32, 16)
    pltpu.sync_copy(bias_h, biasv)
    epsb = [eps0, eps1]
    epdb = [epd0, epd1]
    ssem = [s0, s1]
    dsem = [s2, s3]

    @pl.loop(0, IDXROWS)
    def _ci(i):
        cols[pl.ds(16 * i, 16)] = lane

    for j in range(NBD):
        hbt[j, pl.ds(0, 16)] = jnp.full((16,), DP + _B2H[j], _I32)

    @pl.loop(0, NCH // 2)
    def _chunk(ci):
        w0 = pl.multiple_of(((2 * ci + c) * CH + s * WIN), 16)

        pltpu.async_copy(h_h.at[pl.ds(w0, WIN)], acc, s4).wait()

        @pl.loop(0, WIN)
        def _ir(r):
            dst16 = acc[r, pl.ds(816, 16)]
            e = acc[r, pl.ds(800, 16)] + dst16
            w = jnp.exp(jnp.where(e >= 0.0, e, 0.2 * e))
            dacc[r, pl.ds(0, 16)] = w
            dacc[r, pl.ds(16, 16)] = dst16
            for j in range(NBD):
                wj = jnp.broadcast_to(w[_B2H[j]], (16,))
                acc[r, pl.ds(16 * j, 16)] = acc[r, pl.ds(16 * j, 16)] * wj
            for j in range(NBD, NB):
                acc[r, pl.ds(16 * j, 16)] = jnp.zeros((16,), _F32)

        pltpu.async_copy(src_h.at[pl.ds(0, PIECE)], epsb[0], ssem[0])
        pltpu.async_copy(dst_h.at[pl.ds(0, PIECE)], epdb[0], dsem[0])

        @pl.loop(0, NPIECES // 2)
        def _pp(pp):
            for b in range(2):
                pltpu.make_async_copy(
                    src_h.at[pl.ds(0, PIECE)], epsb[b], ssem[b]).wait()
                pltpu.make_async_copy(
                    dst_h.at[pl.ds(0, PIECE)], epdb[b], dsem[b]).wait()
                nxt = 2 * pp + b + 1

                @pl.when(nxt < NPIECES)
                def _():
                    off = pl.multiple_of(nxt * PIECE, PIECE)
                    pltpu.async_copy(
                        src_h.at[pl.ds(off, PIECE)], epsb[1 - b],
                        ssem[1 - b])
                    pltpu.async_copy(
                        dst_h.at[pl.ds(off, PIECE)], epdb[1 - b],
                        dsem[1 - b])

                eps = epsb[b]
                epd = epdb[b]

                @pl.loop(0, PIECE // 16, init_carry=jnp.int32(0))
                def _scan(v, k):
                    rel = epd[pl.ds(16 * v, 16)] - w0
                    m = (rel >= 0) & (rel < WIN)
                    csum = plsc.cumsum(jnp.where(m, 1, 0).astype(_I32))
                    pos = k + csum - 1
                    plsc.store_scatter(cols, [pos], eps[pl.ds(16 * v, 16)],
                                       mask=m)
                    plsc.store_scatter(cold2, [pos >> 4, pos & 15], rel,
                                       mask=m)
                    return k + csum[15]

                k = _scan
                ng = (k + 15) // 16

                @pl.loop(0, ng)
                def _grp(g):
                    s16 = cols[pl.ds(16 * g, 16)]
                    pltpu.async_copy(h_h.at[s16], rows, s5).wait()
                    rel16 = cold2[g, pl.ds(0, 16)]

                    for r in range(16):
                        @pl.when(16 * g + r < k)
                        def _edge(r=r):
                            rel = rel16[r]
                            e = (rows[r, pl.ds(800, 16)]
                                 + dacc[rel, pl.ds(16, 16)])
                            w = jnp.exp(jnp.where(e >= 0.0, e, 0.2 * e))
                            dacc[rel, pl.ds(0, 16)] = (
                                dacc[rel, pl.ds(0, 16)] + w)
                            biasv[pl.ds(DP, 16)] = w

                            @pl.loop(0, NBD)
                            def _blk(j):
                                hv = hbt[j, pl.ds(0, 16)]
                                wj = plsc.load_gather(biasv, [hv])
                                acc[rel, pl.ds(16 * j, 16)] = (
                                    acc[rel, pl.ds(16 * j, 16)]
                                    + rows[r, pl.ds(16 * j, 16)] * wj)

        @pl.loop(0, WIN)
        def _fr(r):
            d = dacc[r, pl.ds(0, 16)]
            for j in range(NBD):
                dj = jnp.broadcast_to(d[_B2H[j]], (16,))
                v = (acc[r, pl.ds(16 * j, 16)] / dj
                     + biasv[pl.ds(16 * j, 16)])
                acc[r, pl.ds(16 * j, 16)] = jnp.maximum(v, 0.0)

        d1 = pltpu.async_copy(acc, gatr_h.at[pl.ds(w0, WIN)], s4)
        d2 = pltpu.async_copy(dacc, denr_h.at[pl.ds(w0, WIN)], s5)
        d1.wait()
        d2.wait()


def _gat_sc(src, dst, h, bias_pad):
    f = pl.kernel(
        _gat_sc_body,
        out_type=[
            jax.ShapeDtypeStruct((NP, DP), _F32),
            jax.ShapeDtypeStruct((NP, 32), _F32),
        ],
        mesh=_mesh(),
        compiler_params=pltpu.CompilerParams(needs_layout_passes=False),
        scratch_types=[
            pltpu.VMEM((WIN, DP), _F32),
            pltpu.VMEM((WIN, 32), _F32),
            pltpu.VMEM((PIECE,), _I32),
            pltpu.VMEM((PIECE,), _I32),
            pltpu.VMEM((PIECE,), _I32),
            pltpu.VMEM((PIECE,), _I32),
            pltpu.VMEM((PIECE + 32,), _I32),
            pltpu.VMEM((IDXROWS, 16), _I32),
            pltpu.VMEM((16, DP), _F32),
            pltpu.VMEM((BIASW,), _F32),
            pltpu.VMEM((NBD, 16), _I32),
            pltpu.SemaphoreType.DMA,
            pltpu.SemaphoreType.DMA,
            pltpu.SemaphoreType.DMA,
            pltpu.SemaphoreType.DMA,
            pltpu.SemaphoreType.DMA,
            pltpu.SemaphoreType.DMA,
        ],
    )
    return f(src, dst, h, bias_pad)


def _gcn_mm_body(g_ref, w_ref, den_ref, h2_ref):
    h2_ref[...] = jnp.dot(g_ref[...], w_ref[...], preferred_element_type=_F32)
    deg = den_ref[...][:, 10:11]
    h2_ref[:, 800:816] = jnp.broadcast_to(lax.rsqrt(deg), (800, 16))


def _gcn_mm(gatr, w_gcn_pad, denr):
    nblk = NP // 800
    return pl.pallas_call(
        _gcn_mm_body,
        grid=(nblk,),
        in_specs=[
            pl.BlockSpec((800, DP), lambda i: (i, 0)),
            pl.BlockSpec((DP, DP), lambda i: (0, 0)),
            pl.BlockSpec((800, 32), lambda i: (i, 0)),
        ],
        out_specs=pl.BlockSpec((800, DP), lambda i: (i, 0)),
        out_shape=jax.ShapeDtypeStruct((NP, DP), _F32),
    )(gatr, w_gcn_pad, denr)


def _gcn_sc_body(src_h, dst_h, h2_h, bias_h,
                 h3_h,
                 acc, ddw, eps0, epd0, eps1, epd1, cols, cold2, rows,
                 biasv,
                 s0, s1, s2, s3, s4, s5):
    c = lax.axis_index("c")
    s = lax.axis_index("s")
    lane = lax.iota(_I32, 16)
    pltpu.sync_copy(bias_h, biasv)
    epsb = [eps0, eps1]
    epdb = [epd0, epd1]
    ssem = [s0, s1]
    dsem = [s2, s3]

    @pl.loop(0, IDXROWS)
    def _ci(i):
        cols[pl.ds(16 * i, 16)] = lane

    @pl.loop(0, NCH // 2)
    def _chunk(ci):
        w0 = pl.multiple_of(((2 * ci + c) * CH + s * WIN), 16)

        pltpu.async_copy(h2_h.at[pl.ds(w0, WIN)], acc, s4).wait()

        @pl.loop(0, WIN)
        def _ir(r):
            nv = acc[r, pl.ds(800, 16)]
            ddw[r, pl.ds(0, 16)] = nv
            nv2 = nv * nv
            for j in range(NBD):
                acc[r, pl.ds(16 * j, 16)] = acc[r, pl.ds(16 * j, 16)] * nv2
            for j in range(NBD, NB):
                acc[r, pl.ds(16 * j, 16)] = jnp.zeros((16,), _F32)

        pltpu.async_copy(src_h.at[pl.ds(0, PIECE)], epsb[0], ssem[0])
        pltpu.async_copy(dst_h.at[pl.ds(0, PIECE)], epdb[0], dsem[0])

        @pl.loop(0, NPIECES // 2)
        def _pp(pp):
            for b in range(2):
                pltpu.make_async_copy(
                    src_h.at[pl.ds(0, PIECE)], epsb[b], ssem[b]).wait()
                pltpu.make_async_copy(
                    dst_h.at[pl.ds(0, PIECE)], epdb[b], dsem[b]).wait()
                nxt = 2 * pp + b + 1

                @pl.when(nxt < NPIECES)
                def _():
                    off = pl.multiple_of(nxt * PIECE, PIECE)
                    pltpu.async_copy(
                        src_h.at[pl.ds(off, PIECE)], epsb[1 - b],
                        ssem[1 - b])
                    pltpu.async_copy(
                        dst_h.at[pl.ds(off, PIECE)], epdb[1 - b],
                        dsem[1 - b])

                eps = epsb[b]
                epd = epdb[b]

                @pl.loop(0, PIECE // 16, init_carry=jnp.int32(0))
                def _scan(v, k):
                    rel = epd[pl.ds(16 * v, 16)] - w0
                    m = (rel >= 0) & (rel < WIN)
                    csum = plsc.cumsum(jnp.where(m, 1, 0).astype(_I32))
                    pos = k + csum - 1
                    plsc.store_scatter(cols, [pos], eps[pl.ds(16 * v, 16)],
                                       mask=m)
                    plsc.store_scatter(cold2, [pos >> 4, pos & 15], rel,
                                       mask=m)
                    return k + csum[15]

                k = _scan
                ng = (k + 15) // 16

                @pl.loop(0, ng)
                def _grp(g):
                    s16 = cols[pl.ds(16 * g, 16)]
                    pltpu.async_copy(h2_h.at[s16], rows, s5).wait()
                    rel16 = cold2[g, pl.ds(0, 16)]

                    for r in range(16):
                        @pl.when(16 * g + r < k)
                        def _edge(r=r):
                            rel = rel16[r]
                            nv = (rows[r, pl.ds(800, 16)]
                                  * ddw[rel, pl.ds(0, 16)])

                            @pl.loop(0, NBD)
                            def _blk(j):
                                acc[rel, pl.ds(16 * j, 16)] = (
                                    acc[rel, pl.ds(16 * j, 16)]
                                    + rows[r, pl.ds(16 * j, 16)] * nv)

        @pl.loop(0, WIN)
        def _fr(r):
            for j in range(NBD):
                v = acc[r, pl.ds(16 * j, 16)] + biasv[pl.ds(16 * j, 16)]
                acc[r, pl.ds(16 * j, 16)] = jnp.maximum(v, 0.0)

        pltpu.async_copy(acc, h3_h.at[pl.ds(w0, WIN)], s4).wait()


def _gcn_sc(src, dst, h2, bias_pad):
    f = pl.kernel(
        _gcn_sc_body,
        out_type=jax.ShapeDtypeStruct((NP, DP), _F32),
        mesh=_mesh(),
        compiler_params=pltpu.CompilerParams(needs_layout_passes=False),
        scratch_types=[
            pltpu.VMEM((WIN, DP), _F32),
            pltpu.VMEM((WIN, 16), _F32),
            pltpu.VMEM((PIECE,), _I32),
            pltpu.VMEM((PIECE,), _I32),
            pltpu.VMEM((PIECE,), _I32),
            pltpu.VMEM((PIECE,), _I32),
            pltpu.VMEM((PIECE + 32,), _I32),
            pltpu.VMEM((IDXROWS, 16), _I32),
            pltpu.VMEM((16, DP), _F32),
            pltpu.VMEM((BIASW,), _F32),
            pltpu.SemaphoreType.DMA,
            pltpu.SemaphoreType.DMA,
            pltpu.SemaphoreType.DMA,
            pltpu.SemaphoreType.DMA,
            pltpu.SemaphoreType.DMA,
            pltpu.SemaphoreType.DMA,
        ],
    )
    return f(src, dst, h2, bias_pad)


def _pool_body(b_ref, h3_ref, pool_ref, cnt_ref):
    i = pl.program_id(0)

    @pl.when(i == 0)
    def _():
        pool_ref[...] = jnp.zeros_like(pool_ref)
        cnt_ref[...] = jnp.zeros_like(cnt_ref)

    bb = b_ref[...]
    gi = lax.broadcasted_iota(_I32, (1, N_GRAPHS), 1)
    oh = (bb == gi).astype(_F32)
    pool_ref[...] += lax.dot_general(
        oh, h3_ref[...], dimension_numbers=(((0,), (0,)), ((), ())),
        preferred_element_type=_F32)
    cnt_ref[...] += jnp.broadcast_to(
        jnp.sum(oh, axis=0)[:, None], (N_GRAPHS, NW))


def _pool(batch2d, h3):
    nblk = NP // 800
    return pl.pallas_call(
        _pool_body,
        grid=(nblk,),
        in_specs=[
            pl.BlockSpec((800, 1), lambda i: (i, 0)),
            pl.BlockSpec((800, DP), lambda i: (i, 0)),
        ],
        out_specs=[
            pl.BlockSpec((N_GRAPHS, DP), lambda i: (0, 0)),
            pl.BlockSpec((N_GRAPHS, NW), lambda i: (0, 0)),
        ],
        out_shape=[
            jax.ShapeDtypeStruct((N_GRAPHS, DP), _F32),
            jax.ShapeDtypeStruct((N_GRAPHS, NW), _F32),
        ],
    )(batch2d, h3)


def _mlp_body(p_ref, c_ref, w1_ref, b1_ref, w2_ref, b2_ref, out_ref):
    sums = p_ref[...]
    cnt = c_ref[...][:, 0:1]
    mean = sums / jnp.maximum(cnt, 1.0)
    g = jnp.concatenate([mean, sums], axis=1)
    t = jnp.maximum(
        jnp.dot(g, w1_ref[...], preferred_element_type=_F32)
        + b1_ref[...][None, :], 0.0)
    out_ref[...] = (jnp.dot(t, w2_ref[...], preferred_element_type=_F32)
                    + b2_ref[...][None, :])


def _mlp(ps, cnt, W_fc1, b_fc1, W_fc2, b_fc2):
    return pl.pallas_call(
        _mlp_body,
        out_shape=jax.ShapeDtypeStruct((N_GRAPHS, W_fc2.shape[1]), _F32),
    )(ps, cnt, W_fc1, b_fc1, W_fc2, b_fc2)


def _head_pad(v):
    v800 = jnp.pad(v.reshape(HEADS, D_IN), ((0, 0), (0, 2))).reshape(800)
    return jnp.pad(v800, (0, BIASW - 800))


def kernel(x, edge_index, batch, W_gat, a_src, a_dst, b_gat, W_gcn, b_gcn,
           W_fc1, b_fc1, W_fc2, b_fc2):
    src = edge_index[0].astype(_I32)
    dst = edge_index[1].astype(_I32)

    w3 = W_gat.reshape(D_IN, HEADS, D_IN)
    As = jnp.einsum("dhk,hk->dh", w3, a_src)
    Ad = jnp.einsum("dhk,hk->dh", w3, a_dst)
    aa_pad = jnp.zeros((D_IN, NW), _F32)
    aa_pad = aa_pad.at[:, 0:HEADS].set(As).at[:, 16:16 + HEADS].set(Ad)
    w_pad = jnp.pad(
        jnp.pad(w3, ((0, 0), (0, 0), (0, 2))).reshape(D_IN, 800),
        ((0, 0), (0, DP - 800)))
    wg4 = W_gcn.reshape(HEADS, D_IN, HEADS, D_IN)
    wg800 = jnp.pad(wg4, ((0, 0), (0, 2), (0, 0), (0, 2))).reshape(800, 800)
    wg_pad = jnp.pad(wg800, ((0, DP - 800), (0, DP - 800)))
    bgat_pad = _head_pad(b_gat)
    bgcn_pad = _head_pad(b_gcn)
    x_pad = jnp.pad(x, ((0, NP - N_NODES), (0, 0)))
    batch2d = jnp.concatenate([
        batch.astype(_I32),
        jnp.full((NP - N_NODES,), N_GRAPHS, _I32)]).reshape(NP, 1)

    h = _prep(x_pad, w_pad, aa_pad)
    gatr, denr = _gat_sc(src, dst, h, bgat_pad)
    h2 = _gcn_mm(gatr, wg_pad, denr)
    h3 = _gcn_sc(src, dst, h2, bgcn_pad)
    pool, cnt = _pool(batch2d, h3)

    ps = pool[:, :800].reshape(N_GRAPHS, HEADS, D_IN + 2)
    ps = ps[:, :, :D_IN].reshape(N_GRAPHS, D_HID)
    return _mlp(ps, cnt, W_fc1, b_fc1, W_fc2, b_fc2)
